# baseline (speedup 1.0000x reference)
import jax
import jax.numpy as jnp
from jax.experimental import pallas as pl
from jax.experimental.pallas import tpu as pltpu


def _mm_kernel(x_ref, y_ref, o_ref):
    o_ref[0] = jnp.dot(
        x_ref[0],
        y_ref[0],
        preferred_element_type=jnp.float32,
    )


def kernel(Y, x):
    B, N, _ = Y.shape
    _, C, F, _ = x.shape
    M = C * F
    x2 = x.reshape(B, M, N).astype(jnp.bfloat16)
    TN = 1024
    out = pl.pallas_call(
        _mm_kernel,
        grid=(B, N // TN),
        in_specs=[
            pl.BlockSpec((1, M, N), lambda b, j: (b, 0, 0)),
            pl.BlockSpec((1, N, TN), lambda b, j: (b, 0, j)),
        ],
        out_specs=pl.BlockSpec((1, M, TN), lambda b, j: (b, 0, j)),
        out_shape=jax.ShapeDtypeStruct((B, M, N), jnp.float32),
        compiler_params=pltpu.CompilerParams(
            allow_input_fusion=[False, True],
        ),
    )(x2, Y.astype(jnp.bfloat16))
    return out.reshape(B, C, F, N)


# speedup vs baseline: 1.0123x; 1.0123x over previous
import jax
import jax.numpy as jnp
from jax.experimental import pallas as pl
from jax.experimental.pallas import tpu as pltpu


def _mm_kernel(x_ref, y_ref, o_ref):
    o_ref[0] = jnp.dot(
        x_ref[0],
        y_ref[0],
        preferred_element_type=jnp.float32,
    )


def kernel(Y, x):
    B, N, _ = Y.shape
    _, C, F, _ = x.shape
    M = C * F
    x2 = x.reshape(B, M, N).astype(jnp.bfloat16)
    TN = 1024
    out = pl.pallas_call(
        _mm_kernel,
        grid=(B, N // TN),
        in_specs=[
            pl.BlockSpec((1, M, N), lambda b, j: (b, 0, 0)),
            pl.BlockSpec((1, N, TN), lambda b, j: (b, 0, j)),
        ],
        out_specs=pl.BlockSpec((1, M, TN), lambda b, j: (b, 0, j)),
        out_shape=jax.ShapeDtypeStruct((B, M, N), jnp.float32),
        compiler_params=pltpu.CompilerParams(
            allow_input_fusion=[False, True],
            dimension_semantics=("parallel", "parallel"),
        ),
    )(x2, Y.astype(jnp.bfloat16))
    return out.reshape(B, C, F, N)
